# baseline (speedup 1.0000x reference)
import jax
import jax.numpy as jnp
from jax.experimental import pallas as pl
from jax.experimental.pallas import tpu as pltpu

DIM = 2048
BN = 256
J = DIM // BN
NLAYERS = 6


def _celu(x):
    return jnp.where(x > 0, x, jnp.exp(jnp.minimum(x, 0.0)) - 1.0)


def _mlp_kernel(z_ref,
                w0, b0, w1, b1, w2, b2, w3, b3, w4, b4, w5, b5,
                out_ref, h_a, h_b):
    l = pl.program_id(0)
    j = pl.program_id(1)

    @pl.when((l == 0) & (j == 0))
    def _():
        h_a[...] = z_ref[...].reshape(J, BN)

    w_refs = (w0, w1, w2, w3, w4, w5)
    b_refs = (b0, b1, b2, b3, b4, b5)
    for i in range(NLAYERS):
        @pl.when(l == i)
        def _(i=i):
            src = h_a if i % 2 == 0 else h_b
            x = src[...].reshape(1, DIM)
            y = jnp.dot(x, w_refs[i][...], preferred_element_type=jnp.float32)
            y = y + b_refs[i][...]
            if i < NLAYERS - 1:
                y = _celu(y)
                dst = h_b if i % 2 == 0 else h_a
                dst[pl.ds(j, 1), :] = y
            else:
                out_ref[...] = y


def _w_index_map(i):
    def index_map(l, j):
        col = jnp.where(l == i, j, jnp.where(l < i, 0, J - 1))
        return (0, col)
    return index_map


def kernel(z, cur_W0, cur_b0, cur_W1, cur_b1, cur_W2, cur_b2,
           cur_W3, cur_b3, cur_W4, cur_b4, out_W, out_b):
    ws = [cur_W0, cur_W1, cur_W2, cur_W3, cur_W4, out_W]
    bs = [b.reshape(1, DIM) for b in
          (cur_b0, cur_b1, cur_b2, cur_b3, cur_b4, out_b)]

    in_specs = [pl.BlockSpec((1, DIM), lambda l, j: (0, 0))]
    operands = [z]
    for i in range(NLAYERS):
        in_specs.append(pl.BlockSpec((DIM, BN), _w_index_map(i)))
        operands.append(ws[i])
        in_specs.append(pl.BlockSpec((1, BN), lambda l, j: (0, j)))
        operands.append(bs[i])

    out = pl.pallas_call(
        _mlp_kernel,
        grid=(NLAYERS, J),
        in_specs=in_specs,
        out_specs=pl.BlockSpec(
            (1, BN), lambda l, j: (0, jnp.where(l == NLAYERS - 1, j, 0))),
        out_shape=jax.ShapeDtypeStruct((1, DIM), jnp.float32),
        scratch_shapes=[pltpu.VMEM((J, BN), jnp.float32),
                        pltpu.VMEM((J, BN), jnp.float32)],
        compiler_params=pltpu.CompilerParams(
            dimension_semantics=("arbitrary", "arbitrary")),
    )(*operands)
    return out


# speedup vs baseline: 1.1833x; 1.1833x over previous
import jax
import jax.numpy as jnp
from jax.experimental import pallas as pl
from jax.experimental.pallas import tpu as pltpu

DIM = 2048
BN = 512
J = DIM // BN
NLAYERS = 6


def _celu(x):
    return jnp.where(x > 0, x, jnp.exp(jnp.minimum(x, 0.0)) - 1.0)


def _mlp_kernel(z_ref,
                w0, b0, w1, b1, w2, b2, w3, b3, w4, b4, w5, b5,
                out_ref, h_a, h_b):
    l = pl.program_id(0)
    j = pl.program_id(1)

    @pl.when((l == 0) & (j == 0))
    def _():
        h_a[...] = z_ref[...].reshape(J, BN)

    w_refs = (w0, w1, w2, w3, w4, w5)
    b_refs = (b0, b1, b2, b3, b4, b5)
    for i in range(NLAYERS):
        @pl.when(l == i)
        def _(i=i):
            src = h_a if i % 2 == 0 else h_b
            x = src[...].reshape(1, DIM)
            y = jnp.dot(x, w_refs[i][...], preferred_element_type=jnp.float32)
            y = y + b_refs[i][...]
            if i < NLAYERS - 1:
                y = _celu(y)
                dst = h_b if i % 2 == 0 else h_a
                dst[pl.ds(j, 1), :] = y
            else:
                out_ref[...] = y


def _w_index_map(i):
    def index_map(l, j):
        col = jnp.where(l == i, j, jnp.where(l < i, 0, J - 1))
        return (0, col)
    return index_map


def kernel(z, cur_W0, cur_b0, cur_W1, cur_b1, cur_W2, cur_b2,
           cur_W3, cur_b3, cur_W4, cur_b4, out_W, out_b):
    ws = [cur_W0, cur_W1, cur_W2, cur_W3, cur_W4, out_W]
    bs = [b.reshape(1, DIM) for b in
          (cur_b0, cur_b1, cur_b2, cur_b3, cur_b4, out_b)]

    in_specs = [pl.BlockSpec((1, DIM), lambda l, j: (0, 0))]
    operands = [z]
    for i in range(NLAYERS):
        in_specs.append(pl.BlockSpec((DIM, BN), _w_index_map(i)))
        operands.append(ws[i])
        in_specs.append(pl.BlockSpec((1, BN), lambda l, j: (0, j)))
        operands.append(bs[i])

    out = pl.pallas_call(
        _mlp_kernel,
        grid=(NLAYERS, J),
        in_specs=in_specs,
        out_specs=pl.BlockSpec(
            (1, BN), lambda l, j: (0, jnp.where(l == NLAYERS - 1, j, 0))),
        out_shape=jax.ShapeDtypeStruct((1, DIM), jnp.float32),
        scratch_shapes=[pltpu.VMEM((J, BN), jnp.float32),
                        pltpu.VMEM((J, BN), jnp.float32)],
        compiler_params=pltpu.CompilerParams(
            dimension_semantics=("arbitrary", "arbitrary")),
    )(*operands)
    return out
